# Initial kernel scaffold; baseline (speedup 1.0000x reference)
import functools

import jax
import jax.numpy as jnp
from jax import lax
from jax.experimental import pallas as pl
from jax.experimental.pallas import tpu as pltpu
from jax.experimental.pallas import tpu_sc as plsc

B, N, K, D, T = 4096, 50000, 32, 128, 2

_NC, _NS = 2, 16
_NW = _NC * _NS

_TBLK = 2000

_DN = (((1,), (1,)), ((), ()))


def _fuse_tables_body(e0, p0, e1, p1, wf, bf, f0o, f1o):
    a = wf[:, :D]
    bm = wf[:, D:]
    bias = bf[...]
    f0o[...] = jnp.maximum(
        lax.dot_general(e0[...], a, _DN, preferred_element_type=jnp.float32)
        + lax.dot_general(p0[...], bm, _DN, preferred_element_type=jnp.float32)
        + bias, 0.0)
    f1o[...] = jnp.maximum(
        lax.dot_general(e1[...], a, _DN, preferred_element_type=jnp.float32)
        + lax.dot_general(p1[...], bm, _DN, preferred_element_type=jnp.float32)
        + bias, 0.0)


def _fuse_tables(e0, p0, e1, p1, wf, bf2):
    tab_spec = pl.BlockSpec((_TBLK, D), lambda i: (i, 0))
    return pl.pallas_call(
        _fuse_tables_body,
        grid=(N // _TBLK,),
        in_specs=[
            tab_spec, tab_spec, tab_spec, tab_spec,
            pl.BlockSpec((D, 2 * D), lambda i: (0, 0)),
            pl.BlockSpec((1, D), lambda i: (0, 0)),
        ],
        out_specs=[tab_spec, tab_spec],
        out_shape=[
            jax.ShapeDtypeStruct((N, D), jnp.float32),
            jax.ShapeDtypeStruct((N, D), jnp.float32),
        ],
    )(e0, p0, e1, p1, wf, bf2)


_PW = (K * B) // _NW
_C = 512
_NCHUNK = _PW // _C
_PWN = B // _NW


def _gather_body(f0, idx0, f1, idx1, nemb, nprof, nds,
                 out0, out1, one, onp, idxv, rowsv, idxn, rowsn, sem):
    wid = lax.axis_index("s") * _NC + lax.axis_index("c")

    def chunk_loop(tab, idxs, out):
        def body(j, carry):
            base = wid * _PW + j * _C
            pltpu.sync_copy(idxs.at[pl.ds(base, _C)], idxv)
            pltpu.async_copy(tab.at[idxv], rowsv, sem).wait()
            pltpu.sync_copy(rowsv, out.at[pl.ds(base, _C)])
            return carry
        lax.fori_loop(0, _NCHUNK, body, 0)

    chunk_loop(f0, idx0, out0)
    chunk_loop(f1, idx1, out1)

    nb = wid * _PWN
    pltpu.sync_copy(nds.at[pl.ds(nb, _PWN)], idxn)
    pltpu.async_copy(nemb.at[idxn], rowsn, sem).wait()
    pltpu.sync_copy(rowsn, one.at[pl.ds(nb, _PWN)])
    pltpu.async_copy(nprof.at[idxn], rowsn, sem).wait()
    pltpu.sync_copy(rowsn, onp.at[pl.ds(nb, _PWN)])


_gather = functools.partial(
    pl.kernel,
    out_type=[
        jax.ShapeDtypeStruct((K * B, D), jnp.float32),
        jax.ShapeDtypeStruct((K * B, D), jnp.float32),
        jax.ShapeDtypeStruct((B, D), jnp.float32),
        jax.ShapeDtypeStruct((B, D), jnp.float32),
    ],
    mesh=plsc.VectorSubcoreMesh(core_axis_name="c", subcore_axis_name="s"),
    scratch_types=[
        pltpu.VMEM((_C,), jnp.int32),
        pltpu.VMEM((_C, D), jnp.float32),
        pltpu.VMEM((_PWN,), jnp.int32),
        pltpu.VMEM((_PWN, D), jnp.float32),
        pltpu.SemaphoreType.DMA,
    ],
)(_gather_body)


_BB = 256


def _attend_body(ne, npf, nf0, nf1, wf, bf, w1, b1, w2, b2, w, bb, wt,
                 combo, atto):
    a = wf[:, :D]
    bm = wf[:, D:]
    q = jnp.maximum(
        lax.dot_general(ne[...], a, _DN, preferred_element_type=jnp.float32)
        + lax.dot_general(npf[...], bm, _DN, preferred_element_type=jnp.float32)
        + bf[...], 0.0)

    kiota = lax.broadcasted_iota(jnp.int32, (1, K), 1)

    def attend(nf_ref, w1v, b1v):
        s = jnp.zeros((_BB, K), jnp.float32)
        for k in range(K):
            sk = jnp.sum(q * nf_ref[k], axis=1, keepdims=True)
            s = s + sk * (kiota == k).astype(jnp.float32)
        m = jnp.max(s, axis=1, keepdims=True)
        e = jnp.exp(s - m)
        att_k = e / jnp.sum(e, axis=1, keepdims=True)
        feat = jnp.zeros((_BB, D), jnp.float32)
        for k in range(K):
            feat = feat + att_k[:, k:k + 1] * nf_ref[k]
        return jnp.maximum(
            lax.dot_general(feat, w1v, _DN, preferred_element_type=jnp.float32)
            + b1v, 0.0)

    agg0 = attend(nf0, w1[...], b1[...])
    agg1 = attend(nf1, w1[...], b1[...])

    ta = jnp.concatenate([agg0, agg1], axis=1)
    mta = lax.dot_general(ta, wt[...], _DN, preferred_element_type=jnp.float32)
    mm = jnp.max(mta, axis=1, keepdims=True)
    ee = jnp.exp(mta - mm)
    att = ee / jnp.sum(ee, axis=1, keepdims=True)

    fin = att[:, 0:1] * agg0 + att[:, 1:2] * agg1
    fin = jnp.maximum(
        lax.dot_general(fin, w2[...], _DN, preferred_element_type=jnp.float32)
        + b2[...], 0.0)
    comb = jnp.concatenate([q, fin], axis=1)
    combo[...] = jnp.maximum(
        lax.dot_general(comb, w[...], _DN, preferred_element_type=jnp.float32)
        + bb[...], 0.0)
    atto[...] = att


def _attend(ne, npf, nf0, nf1, wf, bf2, w1, b12, w2, b22, w, b2d, wt):
    row_spec = pl.BlockSpec((_BB, D), lambda i: (i, 0))
    nf_spec = pl.BlockSpec((K, _BB, D), lambda i: (0, i, 0))
    full = lambda shape: pl.BlockSpec(shape, lambda i: tuple(0 for _ in shape))
    return pl.pallas_call(
        _attend_body,
        grid=(B // _BB,),
        in_specs=[
            row_spec, row_spec, nf_spec, nf_spec,
            full((D, 2 * D)), full((1, D)),
            full((D, D)), full((1, D)),
            full((D, D)), full((1, D)),
            full((D, 2 * D)), full((1, D)),
            full((T, 2 * D)),
        ],
        out_specs=[row_spec, pl.BlockSpec((_BB, T), lambda i: (i, 0))],
        out_shape=[
            jax.ShapeDtypeStruct((B, D), jnp.float32),
            jax.ShapeDtypeStruct((B, T), jnp.float32),
        ],
    )(ne, npf, nf0, nf1, wf, bf2, w1, b12, w2, b22, w, b2d, wt)


def kernel(nodes, neigh_idx_0, neigh_idx_1, node_emb, node_prof,
           neigh_emb_0, neigh_prof_0, neigh_emb_1, neigh_prof_1,
           Wf, bf, W1, b1, W2, b2, W, b, Wt):
    nodes_i = nodes.astype(jnp.int32)
    idx0t = neigh_idx_0.astype(jnp.int32).T.reshape(-1)
    idx1t = neigh_idx_1.astype(jnp.int32).T.reshape(-1)

    bf2 = bf.reshape(1, D)
    f0, f1 = _fuse_tables(neigh_emb_0, neigh_prof_0, neigh_emb_1,
                          neigh_prof_1, Wf, bf2)
    nf0, nf1, ne, npf = _gather(f0, idx0t, f1, idx1t,
                                node_emb, node_prof, nodes_i)
    comb, att = _attend(ne, npf,
                        nf0.reshape(K, B, D), nf1.reshape(K, B, D),
                        Wf, bf2, W1, b1.reshape(1, D), W2, b2.reshape(1, D),
                        W, b.reshape(1, D), Wt)
    return comb, att.reshape(B, T, 1)


# speedup vs baseline: 7.8612x; 7.8612x over previous
import functools

import jax
import jax.numpy as jnp
from jax import lax
from jax.experimental import pallas as pl
from jax.experimental.pallas import tpu as pltpu
from jax.experimental.pallas import tpu_sc as plsc

B, N, K, D, T = 4096, 50000, 32, 128, 2

_NC, _NS = 2, 16
_NW = _NC * _NS

_TBLK = 2000

_DN = (((1,), (1,)), ((), ()))


def _fuse_tables_body(e0, p0, e1, p1, wf, bf, f0o, f1o):
    a = wf[:, :D]
    bm = wf[:, D:]
    bias = bf[...]
    f0o[...] = jnp.maximum(
        lax.dot_general(e0[...], a, _DN, preferred_element_type=jnp.float32)
        + lax.dot_general(p0[...], bm, _DN, preferred_element_type=jnp.float32)
        + bias, 0.0)
    f1o[...] = jnp.maximum(
        lax.dot_general(e1[...], a, _DN, preferred_element_type=jnp.float32)
        + lax.dot_general(p1[...], bm, _DN, preferred_element_type=jnp.float32)
        + bias, 0.0)


def _fuse_tables(e0, p0, e1, p1, wf, bf2):
    tab_spec = pl.BlockSpec((_TBLK, D), lambda i: (i, 0))
    return pl.pallas_call(
        _fuse_tables_body,
        grid=(N // _TBLK,),
        in_specs=[
            tab_spec, tab_spec, tab_spec, tab_spec,
            pl.BlockSpec((D, 2 * D), lambda i: (0, 0)),
            pl.BlockSpec((1, D), lambda i: (0, 0)),
        ],
        out_specs=[tab_spec, tab_spec],
        out_shape=[
            jax.ShapeDtypeStruct((N, D), jnp.float32),
            jax.ShapeDtypeStruct((N, D), jnp.float32),
        ],
    )(e0, p0, e1, p1, wf, bf2)


_C = 256
_H = 2
_HB = B // _H


def _make_gather_body(pw, nchunk, pwn):
    def _gather_body(f0, idx0, f1, idx1, nemb, nprof, nds,
                     out0, out1, one, onp, idxv0, idxv1, rowsv0, rowsv1,
                     idxn, rowsn, gsem, wsem):
        wid = lax.axis_index("s") * _NC + lax.axis_index("c")
        bufs = ((idxv0, rowsv0), (idxv1, rowsv1))

        def chunk_loop(tab, idxs, out):
            def pair_body(jj, carry):
                for p in range(2):
                    j = jj * 2 + p
                    base = wid * pw + j * _C
                    idxv, rowsv = bufs[p]

                    @pl.when(jj > 0)
                    def _drain():
                        pltpu.make_async_copy(
                            rowsv, out.at[pl.ds(base - 2 * _C, _C)],
                            wsem).wait()

                    pltpu.sync_copy(idxs.at[pl.ds(base, _C)], idxv)
                    pltpu.async_copy(tab.at[idxv], rowsv, gsem).wait()
                    pltpu.async_copy(rowsv, out.at[pl.ds(base, _C)], wsem)
                return carry
            lax.fori_loop(0, nchunk // 2, pair_body, 0)
            for p in range(2):
                base = wid * pw + (nchunk - 2 + p) * _C
                pltpu.make_async_copy(
                    bufs[p][1], out.at[pl.ds(base, _C)], wsem).wait()

        chunk_loop(f0, idx0, out0)
        chunk_loop(f1, idx1, out1)

        nb = wid * pwn
        pltpu.sync_copy(nds.at[pl.ds(nb, pwn)], idxn)
        pltpu.async_copy(nemb.at[idxn], rowsn, gsem).wait()
        pltpu.sync_copy(rowsn, one.at[pl.ds(nb, pwn)])
        pltpu.async_copy(nprof.at[idxn], rowsn, gsem).wait()
        pltpu.sync_copy(rowsn, onp.at[pl.ds(nb, pwn)])
    return _gather_body


@functools.cache
def _build_gather(hb):
    pw = (K * hb) // _NW
    pwn = hb // _NW
    return functools.partial(
        pl.kernel,
        out_type=[
            jax.ShapeDtypeStruct((K * hb, D), jnp.float32),
            jax.ShapeDtypeStruct((K * hb, D), jnp.float32),
            jax.ShapeDtypeStruct((hb, D), jnp.float32),
            jax.ShapeDtypeStruct((hb, D), jnp.float32),
        ],
        mesh=plsc.VectorSubcoreMesh(core_axis_name="c", subcore_axis_name="s"),
        scratch_types=[
            pltpu.VMEM((_C,), jnp.int32),
            pltpu.VMEM((_C,), jnp.int32),
            pltpu.VMEM((_C, D), jnp.float32),
            pltpu.VMEM((_C, D), jnp.float32),
            pltpu.VMEM((pwn,), jnp.int32),
            pltpu.VMEM((pwn, D), jnp.float32),
            pltpu.SemaphoreType.DMA,
            pltpu.SemaphoreType.DMA,
        ],
    )(_make_gather_body(pw, pw // _C, pwn))


_BB = 256


def _attend_body(ne, npf, nf0, nf1, wf, bf, w1, b1, w2, b2, w, bb, wt,
                 combo, atto):
    a = wf[:, :D]
    bm = wf[:, D:]
    q = jnp.maximum(
        lax.dot_general(ne[...], a, _DN, preferred_element_type=jnp.float32)
        + lax.dot_general(npf[...], bm, _DN, preferred_element_type=jnp.float32)
        + bf[...], 0.0)

    ones_dk = jnp.ones((D, K), jnp.float32)
    kiota = lax.broadcasted_iota(jnp.int32, (1, K), 1)
    ones_1d = jnp.ones((1, D), jnp.float32)
    dn_nt = (((1,), (0,)), ((), ()))

    def attend(nf_ref, w1v, b1v):
        s = jnp.zeros((_BB, K), jnp.float32)
        for k in range(K):
            col = lax.dot_general(
                q * nf_ref[k], ones_dk * (kiota == k).astype(jnp.float32),
                dn_nt, preferred_element_type=jnp.float32)
            s = s + col
        m = jnp.max(s, axis=1, keepdims=True)
        e = jnp.exp(s - m)
        att_k = e / jnp.sum(e, axis=1, keepdims=True)
        feat = jnp.zeros((_BB, D), jnp.float32)
        for k in range(K):
            ab = lax.dot_general(att_k[:, k:k + 1], ones_1d, dn_nt,
                                 preferred_element_type=jnp.float32)
            feat = feat + ab * nf_ref[k]
        return jnp.maximum(
            lax.dot_general(feat, w1v, _DN, preferred_element_type=jnp.float32)
            + b1v, 0.0)

    agg0 = attend(nf0, w1[...], b1[...])
    agg1 = attend(nf1, w1[...], b1[...])

    ta = jnp.concatenate([agg0, agg1], axis=1)
    mta = lax.dot_general(ta, wt[...], _DN, preferred_element_type=jnp.float32)
    mm = jnp.max(mta, axis=1, keepdims=True)
    ee = jnp.exp(mta - mm)
    att = ee / jnp.sum(ee, axis=1, keepdims=True)

    fin = att[:, 0:1] * agg0 + att[:, 1:2] * agg1
    fin = jnp.maximum(
        lax.dot_general(fin, w2[...], _DN, preferred_element_type=jnp.float32)
        + b2[...], 0.0)
    comb = jnp.concatenate([q, fin], axis=1)
    combo[...] = jnp.maximum(
        lax.dot_general(comb, w[...], _DN, preferred_element_type=jnp.float32)
        + bb[...], 0.0)
    atto[...] = att


def _attend(ne, npf, nf0, nf1, wf, bf2, w1, b12, w2, b22, w, b2d, wt):
    hb = ne.shape[0]
    row_spec = pl.BlockSpec((_BB, D), lambda i: (i, 0))
    nf_spec = pl.BlockSpec((K, _BB, D), lambda i: (0, i, 0))
    full = lambda shape: pl.BlockSpec(shape, lambda i: tuple(0 for _ in shape))
    return pl.pallas_call(
        _attend_body,
        grid=(hb // _BB,),
        in_specs=[
            row_spec, row_spec, nf_spec, nf_spec,
            full((D, 2 * D)), full((1, D)),
            full((D, D)), full((1, D)),
            full((D, D)), full((1, D)),
            full((D, 2 * D)), full((1, D)),
            full((T, 2 * D)),
        ],
        out_specs=[row_spec, pl.BlockSpec((_BB, T), lambda i: (i, 0))],
        out_shape=[
            jax.ShapeDtypeStruct((hb, D), jnp.float32),
            jax.ShapeDtypeStruct((hb, T), jnp.float32),
        ],
    )(ne, npf, nf0, nf1, wf, bf2, w1, b12, w2, b22, w, b2d, wt)


def kernel(nodes, neigh_idx_0, neigh_idx_1, node_emb, node_prof,
           neigh_emb_0, neigh_prof_0, neigh_emb_1, neigh_prof_1,
           Wf, bf, W1, b1, W2, b2, W, b, Wt):
    nodes_i = nodes.astype(jnp.int32)
    i0 = neigh_idx_0.astype(jnp.int32)
    i1 = neigh_idx_1.astype(jnp.int32)

    bf2 = bf.reshape(1, D)
    f0, f1 = _fuse_tables(neigh_emb_0, neigh_prof_0, neigh_emb_1,
                          neigh_prof_1, Wf, bf2)
    gath = _build_gather(_HB)
    halves = []
    for h in range(_H):
        sl = slice(h * _HB, (h + 1) * _HB)
        halves.append(gath(f0, i0[sl].T.reshape(-1), f1, i1[sl].T.reshape(-1),
                           node_emb, node_prof, nodes_i[sl]))
    combs, atts = [], []
    for h in range(_H):
        nf0, nf1, ne, npf = halves[h]
        comb, att = _attend(ne, npf,
                            nf0.reshape(K, _HB, D), nf1.reshape(K, _HB, D),
                            Wf, bf2, W1, b1.reshape(1, D), W2,
                            b2.reshape(1, D), W, b.reshape(1, D), Wt)
        combs.append(comb)
        atts.append(att)
    comb = jnp.concatenate(combs, axis=0)
    att = jnp.concatenate(atts, axis=0)
    return comb, att.reshape(B, T, 1)
